# Initial kernel scaffold; baseline (speedup 1.0000x reference)
import jax
import jax.numpy as jnp
from jax.experimental import pallas as pl


def kernel(inputs, edge_index, adj_vals, W, b):
    raise NotImplementedError("write your pallas kernel here")


# speedup vs baseline: 2.8005x; 2.8005x over previous
import functools

import jax
import jax.numpy as jnp
from jax import lax
from jax.experimental import pallas as pl
from jax.experimental.pallas import tpu as pltpu
from jax.experimental.pallas import tpu_sc as plsc

NC = 2
NS = 16
LANES = 16
CHUNK = 128


def _matmul_kernel(x_ref, w_ref, o_ref):
    o_ref[...] = jnp.dot(x_ref[...], w_ref[...],
                         preferred_element_type=jnp.float32)


def _post_kernel(y_ref, b_ref, o_ref):
    half = y_ref.shape[-1]
    z0 = jnp.maximum(y_ref[0] + b_ref[0, :half], 0.0)
    z1 = jnp.maximum(y_ref[1] + b_ref[0, half:], 0.0)
    nsq = (jnp.sum(z0 * z0, axis=1, keepdims=True)
           + jnp.sum(z1 * z1, axis=1, keepdims=True))
    inv = 1.0 / jnp.maximum(jnp.sqrt(nsq), 1e-12)
    o_ref[:, :half] = z0 * inv
    o_ref[:, half:] = z1 * inv


def _make_sc_aggregate(n_nodes, half, k_chunks):
    stripe = (pl.cdiv(n_nodes, NS) + 7) // 8 * 8
    n_acc = NS * stripe

    mesh = plsc.VectorSubcoreMesh(core_axis_name="c", subcore_axis_name="s",
                                  num_cores=NC, num_subcores=NS)

    @functools.partial(
        pl.kernel,
        out_type=jax.ShapeDtypeStruct((NC, n_acc, half), jnp.float32),
        mesh=mesh,
        scratch_types=[
            pltpu.VMEM((CHUNK,), jnp.int32),
            pltpu.VMEM((CHUNK,), jnp.int32),
            pltpu.VMEM((CHUNK,), jnp.float32),
            pltpu.VMEM((CHUNK, half), jnp.float32),
            pltpu.VMEM_SHARED((n_acc, half), jnp.float32),
        ],
    )
    def sc_aggregate(x_tab, src_hbm, dst_hbm, val_hbm, out_hbm,
                     srcv, dstv, valv, rows, acc):
        c = lax.axis_index("c")
        s = lax.axis_index("s")

        zero16 = jnp.zeros((LANES,), jnp.float32)

        def zbody(i, _):
            r = i // (half // LANES)
            q = i % (half // LANES)
            rows[r, pl.ds(q * LANES, LANES)] = zero16
            return 0

        lax.fori_loop(0, CHUNK * (half // LANES), zbody, 0)

        full, rem = divmod(stripe, CHUNK)
        for j in range(full):
            pltpu.sync_copy(rows, acc.at[pl.ds(s * stripe + j * CHUNK, CHUNK)])
        if rem:
            pltpu.sync_copy(rows.at[pl.ds(0, rem)],
                            acc.at[pl.ds(s * stripe + full * CHUNK, rem)])
        plsc.subcore_barrier()

        coff = c * n_nodes

        def body(k, _):
            base = (s * k_chunks + k) * CHUNK
            pltpu.sync_copy(src_hbm.at[pl.ds(base, CHUNK)], srcv)
            pltpu.sync_copy(dst_hbm.at[pl.ds(base, CHUNK)], dstv)
            pltpu.sync_copy(val_hbm.at[pl.ds(base, CHUNK)], valv)

            def addoff(i, _):
                srcv[pl.ds(i * LANES, LANES)] = (
                    srcv[pl.ds(i * LANES, LANES)] + coff)
                return 0

            lax.fori_loop(0, CHUNK // LANES, addoff, 0)

            pltpu.sync_copy(x_tab.at[srcv], rows)

            def scale(g, _):
                vg = valv[pl.ds(g * LANES, LANES)]
                for j in range(LANES):
                    e = g * LANES + j
                    v = vg[j]
                    for q in range(half // LANES):
                        rows[e, pl.ds(q * LANES, LANES)] = (
                            rows[e, pl.ds(q * LANES, LANES)] * v)
                return 0

            lax.fori_loop(0, CHUNK // LANES, scale, 0)

            pltpu.sync_copy(rows, acc.at[dstv], add=True)
            return 0

        lax.fori_loop(0, k_chunks, body, 0)
        plsc.subcore_barrier()

        pltpu.sync_copy(acc.at[pl.ds(s * stripe, stripe)],
                        out_hbm.at[c, pl.ds(s * stripe, stripe)])

    return sc_aggregate


def kernel(inputs, edge_index, adj_vals, W, b):
    n_nodes, d_in = inputs.shape
    d_out = W.shape[1]
    n_edges = adj_vals.shape[0]
    half = d_out // 2

    k_chunks = pl.cdiv(n_edges, NS * CHUNK)
    e_pad = NS * CHUNK * k_chunks
    pad = e_pad - n_edges
    dst = edge_index[0]
    src = edge_index[1]
    if pad:
        zi = jnp.zeros((pad,), jnp.int32)
        src = jnp.concatenate([src, zi])
        dst = jnp.concatenate([dst, zi])
        adj_p = jnp.concatenate([adj_vals, jnp.zeros((pad,), jnp.float32)])
    else:
        adj_p = adj_vals

    rb = 1000
    n_rb = n_nodes // rb
    x_tab = pl.pallas_call(
        _matmul_kernel,
        grid=(2, n_rb),
        in_specs=[
            pl.BlockSpec((rb, d_in), lambda h, i: (i, 0)),
            pl.BlockSpec((d_in, half), lambda h, i: (0, h)),
        ],
        out_specs=pl.BlockSpec((rb, half), lambda h, i: (h * n_rb + i, 0)),
        out_shape=jax.ShapeDtypeStruct((2 * n_nodes, half), jnp.float32),
    )(inputs, W)

    y2 = _make_sc_aggregate(n_nodes, half, k_chunks)(x_tab, src, dst, adj_p)
    y2 = y2[:, :n_nodes, :]

    b2 = jnp.broadcast_to(b.reshape(1, d_out), (8, d_out))
    out = pl.pallas_call(
        _post_kernel,
        grid=(n_rb,),
        in_specs=[
            pl.BlockSpec((2, rb, half), lambda i: (0, i, 0)),
            pl.BlockSpec((8, d_out), lambda i: (0, 0)),
        ],
        out_specs=pl.BlockSpec((rb, d_out), lambda i: (i, 0)),
        out_shape=jax.ShapeDtypeStruct((n_nodes, d_out), jnp.float32),
    )(y2, b2)
    return out


# speedup vs baseline: 3.0875x; 1.1025x over previous
import functools

import jax
import jax.numpy as jnp
from jax import lax
from jax.experimental import pallas as pl
from jax.experimental.pallas import tpu as pltpu
from jax.experimental.pallas import tpu_sc as plsc

NC = 2
NS = 16
LANES = 16
CHUNK = 128
RB = 3


def _matmul_kernel(x_ref, w_ref, o_ref):
    o_ref[...] = jnp.dot(x_ref[...], w_ref[...],
                         preferred_element_type=jnp.float32)


def _post_kernel(y_ref, b_ref, o_ref):
    half = y_ref.shape[-1]
    z0 = jnp.maximum(y_ref[0] + b_ref[0, :half], 0.0)
    z1 = jnp.maximum(y_ref[1] + b_ref[0, half:], 0.0)
    nsq = (jnp.sum(z0 * z0, axis=1, keepdims=True)
           + jnp.sum(z1 * z1, axis=1, keepdims=True))
    inv = 1.0 / jnp.maximum(jnp.sqrt(nsq), 1e-12)
    o_ref[:, :half] = z0 * inv
    o_ref[:, half:] = z1 * inv


def _make_sc_aggregate(n_nodes, half, k_chunks, e_pad):
    stripe = (pl.cdiv(n_nodes, NS) + 7) // 8 * 8
    n_acc = NS * stripe

    mesh = plsc.VectorSubcoreMesh(core_axis_name="c", subcore_axis_name="s",
                                  num_cores=NC, num_subcores=NS)

    @functools.partial(
        pl.kernel,
        out_type=jax.ShapeDtypeStruct((NC, n_acc, half), jnp.float32),
        mesh=mesh,
        scratch_types=[
            pltpu.VMEM((4, CHUNK), jnp.int32),
            pltpu.VMEM((4, CHUNK), jnp.int32),
            pltpu.VMEM((4, CHUNK), jnp.float32),
            pltpu.VMEM((2, CHUNK, half), jnp.float32),
            pltpu.VMEM_SHARED((n_acc, half), jnp.float32),
            pltpu.SemaphoreType.DMA((2,)),
            pltpu.SemaphoreType.DMA((2,)),
            pltpu.SemaphoreType.DMA((4,)),
        ],
    )
    def sc_aggregate(src_hbm, dst_hbm, val_hbm, x_tab, out_hbm,
                     srcv, dstv, valv, rows, acc, gsem, ssem, isem):
        c = lax.axis_index("c")
        s = lax.axis_index("s")

        zero16 = jnp.zeros((LANES,), jnp.float32)

        def zbody(i, _):
            r = i // (half // LANES)
            q = i % (half // LANES)
            rows[0, r, pl.ds(q * LANES, LANES)] = zero16
            return 0

        lax.fori_loop(0, CHUNK * (half // LANES), zbody, 0)

        zsrc = rows.at[0]
        full, rem = divmod(stripe, CHUNK)
        for j in range(full):
            pltpu.sync_copy(zsrc, acc.at[pl.ds(s * stripe + j * CHUNK, CHUNK)])
        if rem:
            pltpu.sync_copy(zsrc.at[pl.ds(0, rem)],
                            acc.at[pl.ds(s * stripe + full * CHUNK, rem)])
        plsc.subcore_barrier()

        ebase = s * k_chunks * CHUNK
        src_off = c * e_pad + ebase

        def issue_idx(k, m):
            base = ebase + k * CHUNK
            sbase = src_off + k * CHUNK
            return (
                pltpu.async_copy(src_hbm.at[pl.ds(sbase, CHUNK)],
                                 srcv.at[m], isem.at[m]),
                pltpu.async_copy(dst_hbm.at[pl.ds(base, CHUNK)],
                                 dstv.at[m], isem.at[m]),
                pltpu.async_copy(val_hbm.at[pl.ds(base, CHUNK)],
                                 valv.at[m], isem.at[m]),
            )

        def wait_idx(k, m):
            base = ebase + k * CHUNK
            sbase = src_off + k * CHUNK
            pltpu.make_async_copy(src_hbm.at[pl.ds(sbase, CHUNK)],
                                  srcv.at[m], isem.at[m]).wait()
            pltpu.make_async_copy(dst_hbm.at[pl.ds(base, CHUNK)],
                                  dstv.at[m], isem.at[m]).wait()
            pltpu.make_async_copy(val_hbm.at[pl.ds(base, CHUNK)],
                                  valv.at[m], isem.at[m]).wait()

        def issue_gather(h, m):
            pltpu.async_copy(x_tab.at[srcv.at[m]], rows.at[h], gsem.at[h])

        def wait_gather(h, m):
            pltpu.make_async_copy(x_tab.at[srcv.at[m]], rows.at[h],
                                  gsem.at[h]).wait()

        def issue_scatter(h, m):
            pltpu.async_copy(rows.at[h], acc.at[dstv.at[m]], ssem.at[h],
                             add=True)

        def wait_scatter(h, m):
            pltpu.make_async_copy(rows.at[h], acc.at[dstv.at[m]],
                                  ssem.at[h]).wait()

        def scale_chunk(h, m):
            def scale(g, _):
                vg = valv[m, pl.ds(g * LANES, LANES)]
                for j in range(LANES):
                    e = g * LANES + j
                    v = vg[j]
                    for q in range(half // LANES):
                        rows[h, e, pl.ds(q * LANES, LANES)] = (
                            rows[h, e, pl.ds(q * LANES, LANES)] * v)
                return 0

            lax.fori_loop(0, CHUNK // LANES, scale, 0)

        def step(k, m, first=False, with_d=True, with_f=True):
            h = m % 2
            other = 1 - h
            m1 = (m + 1) % 4
            m2 = (m + 2) % 4
            mprev = (m - 1) % 4
            wait_gather(h, m)
            scale_chunk(h, m)
            issue_scatter(h, m)
            if with_d:
                issue_idx(k + 2, m2)
            if not first:
                wait_scatter(other, mprev)
            if with_f:
                wait_idx(k + 1, m1)
                issue_gather(other, m1)

        for d in issue_idx(0, 0):
            d.wait()
        issue_gather(0, 0)
        issue_idx(1, 1)
        step(0, 0, first=True)

        def body(i, _):
            k = 4 * i + 1
            step(k, 1)
            step(k + 1, 2)
            step(k + 2, 3)
            step(k + 3, 0)
            return 0

        lax.fori_loop(0, (k_chunks - 4) // 4, body, 0)
        kk = k_chunks
        step(kk - 3, (kk - 3) % 4)
        step(kk - 2, (kk - 2) % 4, with_d=False)
        step(kk - 1, (kk - 1) % 4, with_d=False, with_f=False)
        wait_scatter((kk - 1) % 2, (kk - 1) % 4)

        plsc.subcore_barrier()

        pltpu.sync_copy(acc.at[pl.ds(s * stripe, stripe)],
                        out_hbm.at[c, pl.ds(s * stripe, stripe)])

    return sc_aggregate


def kernel(inputs, edge_index, adj_vals, W, b):
    n_nodes, d_in = inputs.shape
    d_out = W.shape[1]
    n_edges = adj_vals.shape[0]
    half = d_out // 2

    k_chunks = max(8, ((pl.cdiv(n_edges, NS * CHUNK) + 3) // 4) * 4)
    e_pad = NS * CHUNK * k_chunks
    pad = e_pad - n_edges
    dst = edge_index[0]
    src = edge_index[1]
    if pad:
        zi = jnp.zeros((pad,), jnp.int32)
        src = jnp.concatenate([src, zi])
        dst = jnp.concatenate([dst, zi])
        adj_p = jnp.concatenate([adj_vals, jnp.zeros((pad,), jnp.float32)])
    else:
        adj_p = adj_vals
    src2 = jnp.concatenate([src, src + n_nodes])

    rb = 1000
    n_rb = n_nodes // rb
    x_tab = pl.pallas_call(
        _matmul_kernel,
        grid=(2, n_rb),
        in_specs=[
            pl.BlockSpec((rb, d_in), lambda h, i: (i, 0)),
            pl.BlockSpec((d_in, half), lambda h, i: (0, h)),
        ],
        out_specs=pl.BlockSpec((rb, half), lambda h, i: (h * n_rb + i, 0)),
        out_shape=jax.ShapeDtypeStruct((2 * n_nodes, half), jnp.float32),
    )(inputs, W)

    y2 = _make_sc_aggregate(n_nodes, half, k_chunks, e_pad)(
        src2, dst, adj_p, x_tab)
    y2 = y2[:, :n_nodes, :]

    b2 = jnp.broadcast_to(b.reshape(1, d_out), (8, d_out))
    out = pl.pallas_call(
        _post_kernel,
        grid=(n_rb,),
        in_specs=[
            pl.BlockSpec((2, rb, half), lambda i: (0, i, 0)),
            pl.BlockSpec((8, d_out), lambda i: (0, 0)),
        ],
        out_specs=pl.BlockSpec((rb, d_out), lambda i: (i, 0)),
        out_shape=jax.ShapeDtypeStruct((n_nodes, d_out), jnp.float32),
    )(y2, b2)
    return out


# speedup vs baseline: 3.4240x; 1.1090x over previous
import functools

import jax
import jax.numpy as jnp
from jax import lax
from jax.experimental import pallas as pl
from jax.experimental.pallas import tpu as pltpu
from jax.experimental.pallas import tpu_sc as plsc

NC = 2
NS = 16
LANES = 16
CHUNK = 128


def _matmul_kernel(x_ref, w_ref, o_ref):
    o_ref[...] = jnp.dot(x_ref[...], w_ref[...],
                         preferred_element_type=jnp.float32)


def _post_kernel(y_ref, b_ref, o_ref):
    half = y_ref.shape[-1]
    z0 = jnp.maximum(y_ref[0] + b_ref[0, :half], 0.0)
    z1 = jnp.maximum(y_ref[1] + b_ref[0, half:], 0.0)
    nsq = (jnp.sum(z0 * z0, axis=1, keepdims=True)
           + jnp.sum(z1 * z1, axis=1, keepdims=True))
    inv = 1.0 / jnp.maximum(jnp.sqrt(nsq), 1e-12)
    o_ref[:, :half] = z0 * inv
    o_ref[:, half:] = z1 * inv


def _make_sc_aggregate(n_nodes, half, k_chunks, e_pad):
    stripe = (pl.cdiv(n_nodes, NS) + 7) // 8 * 8
    n_acc = NS * stripe

    mesh = plsc.VectorSubcoreMesh(core_axis_name="c", subcore_axis_name="s",
                                  num_cores=NC, num_subcores=NS)

    @functools.partial(
        pl.kernel,
        out_type=jax.ShapeDtypeStruct((NC, n_acc, half), jnp.float32),
        mesh=mesh,
        scratch_types=[
            pltpu.VMEM((4, CHUNK), jnp.int32),
            pltpu.VMEM((4, CHUNK), jnp.int32),
            pltpu.VMEM((4, CHUNK), jnp.float32),
            pltpu.VMEM((2, CHUNK, half), jnp.float32),
            pltpu.VMEM_SHARED((n_acc, half), jnp.float32),
            pltpu.SemaphoreType.DMA((2,)),
            pltpu.SemaphoreType.DMA((2,)),
            pltpu.SemaphoreType.DMA((4,)),
        ],
    )
    def sc_aggregate(src_hbm, dst_hbm, val_hbm, x_tab, out_hbm,
                     srcv, dstv, valv, rows, acc, gsem, ssem, isem):
        c = lax.axis_index("c")
        s = lax.axis_index("s")

        zero16 = jnp.zeros((LANES,), jnp.float32)

        def zbody(i, _):
            r = i // (half // LANES)
            q = i % (half // LANES)
            rows[0, r, pl.ds(q * LANES, LANES)] = zero16
            return 0

        lax.fori_loop(0, CHUNK * (half // LANES), zbody, 0)

        zsrc = rows.at[0]
        full, rem = divmod(stripe, CHUNK)
        for j in range(full):
            pltpu.sync_copy(zsrc, acc.at[pl.ds(s * stripe + j * CHUNK, CHUNK)])
        if rem:
            pltpu.sync_copy(zsrc.at[pl.ds(0, rem)],
                            acc.at[pl.ds(s * stripe + full * CHUNK, rem)])
        plsc.subcore_barrier()

        ebase = s * k_chunks * CHUNK
        src_off = c * e_pad + ebase

        def issue_idx(k, m):
            base = ebase + k * CHUNK
            sbase = src_off + k * CHUNK
            return (
                pltpu.async_copy(src_hbm.at[pl.ds(sbase, CHUNK)],
                                 srcv.at[m], isem.at[m]),
                pltpu.async_copy(dst_hbm.at[pl.ds(base, CHUNK)],
                                 dstv.at[m], isem.at[m]),
                pltpu.async_copy(val_hbm.at[pl.ds(base, CHUNK)],
                                 valv.at[m], isem.at[m]),
            )

        def wait_idx(k, m):
            base = ebase + k * CHUNK
            sbase = src_off + k * CHUNK
            pltpu.make_async_copy(src_hbm.at[pl.ds(sbase, CHUNK)],
                                  srcv.at[m], isem.at[m]).wait()
            pltpu.make_async_copy(dst_hbm.at[pl.ds(base, CHUNK)],
                                  dstv.at[m], isem.at[m]).wait()
            pltpu.make_async_copy(val_hbm.at[pl.ds(base, CHUNK)],
                                  valv.at[m], isem.at[m]).wait()

        def issue_gather(h, m):
            pltpu.async_copy(x_tab.at[srcv.at[m]], rows.at[h], gsem.at[h])

        def wait_gather(h, m):
            pltpu.make_async_copy(x_tab.at[srcv.at[m]], rows.at[h],
                                  gsem.at[h]).wait()

        def issue_scatter(h, m):
            pltpu.async_copy(rows.at[h], acc.at[dstv.at[m]], ssem.at[h],
                             add=True)

        def wait_scatter(h, m):
            pltpu.make_async_copy(rows.at[h], acc.at[dstv.at[m]],
                                  ssem.at[h]).wait()

        def scale_chunk(h, m):
            def scale(g, _):
                vg = valv[m, pl.ds(g * LANES, LANES)]
                for j in range(LANES):
                    e = g * LANES + j
                    v = vg[j]
                    for q in range(half // LANES):
                        rows[h, e, pl.ds(q * LANES, LANES)] = (
                            rows[h, e, pl.ds(q * LANES, LANES)] * v)
                return 0

            lax.fori_loop(0, CHUNK // LANES, scale, 0)

        def step(k, m, first=False, with_d=True, with_f=True):
            h = m % 2
            other = 1 - h
            m1 = (m + 1) % 4
            m2 = (m + 2) % 4
            mprev = (m - 1) % 4
            wait_gather(h, m)
            if not first:
                wait_scatter(other, mprev)
            if with_f:
                wait_idx(k + 1, m1)
                issue_gather(other, m1)
            scale_chunk(h, m)
            issue_scatter(h, m)
            if with_d:
                issue_idx(k + 2, m2)

        for d in issue_idx(0, 0):
            d.wait()
        issue_gather(0, 0)
        issue_idx(1, 1)
        step(0, 0, first=True)

        def body(i, _):
            k = 4 * i + 1
            step(k, 1)
            step(k + 1, 2)
            step(k + 2, 3)
            step(k + 3, 0)
            return 0

        lax.fori_loop(0, (k_chunks - 4) // 4, body, 0)
        kk = k_chunks
        step(kk - 3, (kk - 3) % 4)
        step(kk - 2, (kk - 2) % 4, with_d=False)
        step(kk - 1, (kk - 1) % 4, with_d=False, with_f=False)
        wait_scatter((kk - 1) % 2, (kk - 1) % 4)

        plsc.subcore_barrier()

        pltpu.sync_copy(acc.at[pl.ds(s * stripe, stripe)],
                        out_hbm.at[c, pl.ds(s * stripe, stripe)])

    return sc_aggregate


def kernel(inputs, edge_index, adj_vals, W, b):
    n_nodes, d_in = inputs.shape
    d_out = W.shape[1]
    n_edges = adj_vals.shape[0]
    half = d_out // 2

    k_chunks = max(8, ((pl.cdiv(n_edges, NS * CHUNK) + 3) // 4) * 4)
    e_pad = NS * CHUNK * k_chunks
    pad = e_pad - n_edges
    dst = edge_index[0]
    src = edge_index[1]
    if pad:
        zi = jnp.zeros((pad,), jnp.int32)
        src = jnp.concatenate([src, zi])
        dst = jnp.concatenate([dst, zi])
        adj_p = jnp.concatenate([adj_vals, jnp.zeros((pad,), jnp.float32)])
    else:
        adj_p = adj_vals
    src2 = jnp.concatenate([src, src + n_nodes])

    rb = 1000
    n_rb = n_nodes // rb
    x_tab = pl.pallas_call(
        _matmul_kernel,
        grid=(2, n_rb),
        in_specs=[
            pl.BlockSpec((rb, d_in), lambda h, i: (i, 0)),
            pl.BlockSpec((d_in, half), lambda h, i: (0, h)),
        ],
        out_specs=pl.BlockSpec((rb, half), lambda h, i: (h * n_rb + i, 0)),
        out_shape=jax.ShapeDtypeStruct((2 * n_nodes, half), jnp.float32),
    )(inputs, W)

    y2 = _make_sc_aggregate(n_nodes, half, k_chunks, e_pad)(
        src2, dst, adj_p, x_tab)
    y2 = y2[:, :n_nodes, :]

    b2 = jnp.broadcast_to(b.reshape(1, d_out), (8, d_out))
    out = pl.pallas_call(
        _post_kernel,
        grid=(n_rb,),
        in_specs=[
            pl.BlockSpec((2, rb, half), lambda i: (0, i, 0)),
            pl.BlockSpec((8, d_out), lambda i: (0, 0)),
        ],
        out_specs=pl.BlockSpec((rb, d_out), lambda i: (i, 0)),
        out_shape=jax.ShapeDtypeStruct((n_nodes, d_out), jnp.float32),
    )(y2, b2)
    return out


# speedup vs baseline: 3.4587x; 1.0101x over previous
import functools

import jax
import jax.numpy as jnp
from jax import lax
from jax.experimental import pallas as pl
from jax.experimental.pallas import tpu as pltpu
from jax.experimental.pallas import tpu_sc as plsc

NC = 2
NS = 16
LANES = 16
CHUNK = 128


def _matmul_kernel(x_ref, w_ref, o_ref):
    o_ref[...] = jnp.dot(x_ref[...], w_ref[...],
                         preferred_element_type=jnp.float32)


def _post_kernel(y_ref, b_ref, o_ref):
    half = y_ref.shape[-1]
    z0 = jnp.maximum(y_ref[0] + b_ref[0, :half], 0.0)
    z1 = jnp.maximum(y_ref[1] + b_ref[0, half:], 0.0)
    nsq = (jnp.sum(z0 * z0, axis=1, keepdims=True)
           + jnp.sum(z1 * z1, axis=1, keepdims=True))
    inv = 1.0 / jnp.maximum(jnp.sqrt(nsq), 1e-12)
    o_ref[:, :half] = z0 * inv
    o_ref[:, half:] = z1 * inv


def _make_sc_aggregate(n_nodes, half, k_chunks, e_pad):
    stripe = (pl.cdiv(n_nodes, NS) + 7) // 8 * 8
    n_acc = NS * stripe

    mesh = plsc.VectorSubcoreMesh(core_axis_name="c", subcore_axis_name="s",
                                  num_cores=NC, num_subcores=NS)

    @functools.partial(
        pl.kernel,
        out_type=jax.ShapeDtypeStruct((NC, n_acc, half), jnp.float32),
        mesh=mesh,
        scratch_types=[
            pltpu.VMEM((4, CHUNK), jnp.int32),
            pltpu.VMEM((4, CHUNK), jnp.int32),
            pltpu.VMEM((4, CHUNK), jnp.float32),
            pltpu.VMEM((2, CHUNK, half), jnp.float32),
            pltpu.VMEM_SHARED((n_acc, half), jnp.float32),
            pltpu.SemaphoreType.DMA((2,)),
            pltpu.SemaphoreType.DMA((2,)),
            pltpu.SemaphoreType.DMA((4,)),
        ],
    )
    def sc_aggregate(src_hbm, dst_hbm, val_hbm, x_tab, out_hbm,
                     srcv, dstv, valv, rows, acc, gsem, ssem, isem):
        c = lax.axis_index("c")
        s = lax.axis_index("s")

        zero16 = jnp.zeros((LANES,), jnp.float32)

        def zbody(i, _):
            r = i // (half // LANES)
            q = i % (half // LANES)
            rows[0, r, pl.ds(q * LANES, LANES)] = zero16
            return 0

        lax.fori_loop(0, CHUNK * (half // LANES), zbody, 0)

        zsrc = rows.at[0]
        full, rem = divmod(stripe, CHUNK)
        for j in range(full):
            pltpu.sync_copy(zsrc, acc.at[pl.ds(s * stripe + j * CHUNK, CHUNK)])
        if rem:
            pltpu.sync_copy(zsrc.at[pl.ds(0, rem)],
                            acc.at[pl.ds(s * stripe + full * CHUNK, rem)])
        plsc.subcore_barrier()

        ebase = s * k_chunks * CHUNK
        src_off = c * e_pad + ebase

        def issue_idx(k, m):
            base = ebase + k * CHUNK
            sbase = src_off + k * CHUNK
            return (
                pltpu.async_copy(src_hbm.at[pl.ds(sbase, CHUNK)],
                                 srcv.at[m], isem.at[m]),
                pltpu.async_copy(dst_hbm.at[pl.ds(base, CHUNK)],
                                 dstv.at[m], isem.at[m]),
                pltpu.async_copy(val_hbm.at[pl.ds(base, CHUNK)],
                                 valv.at[m], isem.at[m]),
            )

        def wait_idx(k, m):
            base = ebase + k * CHUNK
            sbase = src_off + k * CHUNK
            pltpu.make_async_copy(src_hbm.at[pl.ds(sbase, CHUNK)],
                                  srcv.at[m], isem.at[m]).wait()
            pltpu.make_async_copy(dst_hbm.at[pl.ds(base, CHUNK)],
                                  dstv.at[m], isem.at[m]).wait()
            pltpu.make_async_copy(val_hbm.at[pl.ds(base, CHUNK)],
                                  valv.at[m], isem.at[m]).wait()

        def issue_gather(h, m):
            pltpu.async_copy(x_tab.at[srcv.at[m]], rows.at[h], gsem.at[h])

        def wait_gather(h, m):
            pltpu.make_async_copy(x_tab.at[srcv.at[m]], rows.at[h],
                                  gsem.at[h]).wait()

        def issue_scatter(h, m):
            pltpu.async_copy(rows.at[h], acc.at[dstv.at[m]], ssem.at[h],
                             add=True)

        def wait_scatter(h, m):
            pltpu.make_async_copy(rows.at[h], acc.at[dstv.at[m]],
                                  ssem.at[h]).wait()

        def scale_chunk(h, m):
            def scale(g, _):
                vg = valv[m, pl.ds(g * LANES, LANES)]
                for j in range(LANES):
                    e = g * LANES + j
                    v = vg[j]
                    for q in range(half // LANES):
                        rows[h, e, pl.ds(q * LANES, LANES)] = (
                            rows[h, e, pl.ds(q * LANES, LANES)] * v)
                return 0

            lax.fori_loop(0, CHUNK // LANES, scale, 0)

        def step(k, m, first=False, with_d=True, with_f=True):
            h = m % 2
            other = 1 - h
            m1 = (m + 1) % 4
            m2 = (m + 2) % 4
            mprev = (m - 1) % 4
            wait_gather(h, m)
            if not first:
                wait_scatter(other, mprev)
            if with_f:
                wait_idx(k + 1, m1)
                issue_gather(other, m1)
            scale_chunk(h, m)
            issue_scatter(h, m)
            if with_d:
                issue_idx(k + 2, m2)

        for d in issue_idx(0, 0):
            d.wait()
        issue_gather(0, 0)
        issue_idx(1, 1)
        step(0, 0, first=True)

        def body(i, _):
            k = 4 * i + 1
            step(k, 1)
            step(k + 1, 2)
            step(k + 2, 3)
            step(k + 3, 0)
            return 0

        lax.fori_loop(0, (k_chunks - 4) // 4, body, 0)
        kk = k_chunks
        step(kk - 3, (kk - 3) % 4)
        step(kk - 2, (kk - 2) % 4, with_d=False)
        step(kk - 1, (kk - 1) % 4, with_d=False, with_f=False)
        wait_scatter((kk - 1) % 2, (kk - 1) % 4)

        plsc.subcore_barrier()

        pltpu.sync_copy(acc.at[pl.ds(s * stripe, stripe)],
                        out_hbm.at[c, pl.ds(s * stripe, stripe)])

    return sc_aggregate


def kernel(inputs, edge_index, adj_vals, W, b):
    n_nodes, d_in = inputs.shape
    d_out = W.shape[1]
    n_edges = adj_vals.shape[0]
    half = d_out // 2

    k_chunks = max(8, ((pl.cdiv(n_edges, NS * CHUNK) + 3) // 4) * 4)
    e_pad = NS * CHUNK * k_chunks
    pad = e_pad - n_edges
    dst = edge_index[0]
    src = edge_index[1]
    if pad:
        zi = jnp.zeros((pad,), jnp.int32)
        src = jnp.concatenate([src, zi])
        dst = jnp.concatenate([dst, zi])
        adj_p = jnp.concatenate([adj_vals, jnp.zeros((pad,), jnp.float32)])
    else:
        adj_p = adj_vals
    src2 = jnp.concatenate([src, src + n_nodes])

    rb = 1000
    n_rb = n_nodes // rb
    x_tab = pl.pallas_call(
        _matmul_kernel,
        grid=(2, n_rb),
        in_specs=[
            pl.BlockSpec((rb, d_in), lambda h, i: (i, 0)),
            pl.BlockSpec((d_in, half), lambda h, i: (0, h)),
        ],
        out_specs=pl.BlockSpec((rb, half), lambda h, i: (h * n_rb + i, 0)),
        out_shape=jax.ShapeDtypeStruct((2 * n_nodes, half), jnp.float32),
    )(inputs, W)

    y2 = _make_sc_aggregate(n_nodes, half, k_chunks, e_pad)(
        src2, dst, adj_p, x_tab)

    b2 = jnp.broadcast_to(b.reshape(1, d_out), (8, d_out))
    out = pl.pallas_call(
        _post_kernel,
        grid=(n_rb,),
        in_specs=[
            pl.BlockSpec((2, rb, half), lambda i: (0, i, 0)),
            pl.BlockSpec((8, d_out), lambda i: (0, 0)),
        ],
        out_specs=pl.BlockSpec((rb, d_out), lambda i: (i, 0)),
        out_shape=jax.ShapeDtypeStruct((n_nodes, d_out), jnp.float32),
    )(y2, b2)
    return out
